# baseline (speedup 1.0000x reference)
import functools

import jax
import jax.numpy as jnp
from jax import lax
from jax.experimental import pallas as pl
from jax.experimental.pallas import tpu as pltpu
from jax.experimental.pallas import tpu_sc as plsc

D = 64
SEQ = 200
L = 16
NV = D // L
NC = 2
NS = 16
NW = NC * NS
BW = 128


def _hsum16(v):
    lanes = lax.iota(jnp.int32, L)
    dnums = lax.GatherDimensionNumbers(
        offset_dims=(), collapsed_slice_dims=(0,), start_index_map=(0,))
    for sh in (8, 4, 2, 1):
        perm = lax.bitwise_xor(lanes, jnp.int32(sh))
        v = v + lax.gather(v, perm[:, None], dnums, slice_sizes=(1,),
                           mode=lax.GatherScatterMode.PROMISE_IN_BOUNDS)
    return v


def _rsqrt16(v):
    i = lax.bitcast_convert_type(v, jnp.int32)
    i = jnp.int32(0x5F3759DF) - lax.shift_right_logical(i, 1)
    y = lax.bitcast_convert_type(i, jnp.float32)
    for _ in range(2):
        y = y * (1.5 - 0.5 * v * y * y)
    return y


@functools.lru_cache(maxsize=None)
def _make_sc_embed(batch):
    mesh = plsc.VectorSubcoreMesh(core_axis_name="c", subcore_axis_name="s")

    @functools.partial(
        pl.kernel,
        out_type=jax.ShapeDtypeStruct((SEQ, D, batch), jnp.float32),
        mesh=mesh,
        compiler_params=pltpu.CompilerParams(
            use_tc_tiling_on_sc=False, needs_layout_passes=False),
        scratch_types=[
            pltpu.VMEM((SEQ, BW), jnp.int32),
            pltpu.VMEM((D, 512), jnp.float32),
            pltpu.VMEM((D,), jnp.float32),
            pltpu.VMEM((D,), jnp.float32),
            pltpu.VMEM((BW, D), jnp.float32),
            pltpu.VMEM((D, BW), jnp.float32),
            pltpu.SemaphoreType.DMA,
        ],
    )
    def sc_embed(idt_hbm, wt_hbm, post_hbm, sc_hbm, sh_hbm, out_hbm,
                 ids_v, post_v, scale_v, shift_v, rows_v, outb_v, sem):
        wid = lax.axis_index("s") * NC + lax.axis_index("c")
        wb = wid * BW
        pltpu.sync_copy(idt_hbm.at[:, pl.ds(wb, BW)], ids_v)
        pltpu.sync_copy(post_hbm, post_v)
        pltpu.sync_copy(sc_hbm, scale_v)
        pltpu.sync_copy(sh_hbm, shift_v)

        scs = [scale_v[pl.ds(q * L, L)] for q in range(NV)]
        shs = [shift_v[pl.ds(q * L, L)] for q in range(NV)]
        lanes = lax.iota(jnp.int32, L)

        def chunk(t, carry):
            pltpu.async_copy(wt_hbm.at[ids_v.at[t]], rows_v, sem).wait()

            tv = lax.broadcast(t, (L,))
            pe = [plsc.load_gather(post_v, [q * L + lanes, tv])
                  for q in range(NV)]

            @plsc.parallel_loop(0, BW, step=1, unroll=8)
            def _row(j):
                e = [rows_v[j, pl.ds(q * L, L)] + pe[q] for q in range(NV)]
                s = (e[0] + e[1]) + (e[2] + e[3])
                q2 = ((e[0] * e[0] + e[1] * e[1])
                      + (e[2] * e[2] + e[3] * e[3]))
                mu = _hsum16(s) * (1.0 / D)
                exx = _hsum16(q2) * (1.0 / D)
                y = _rsqrt16(exx - mu * mu + 1e-12)
                jv = lax.broadcast(j, (L,))
                for q in range(NV):
                    plsc.store_scatter(
                        outb_v, [q * L + lanes, jv],
                        ((e[q] - mu) * y) * scs[q] + shs[q])

            pltpu.sync_copy(outb_v, out_hbm.at[t, :, pl.ds(wb, BW)])
            return carry

        lax.fori_loop(0, SEQ, chunk, 0)

    return sc_embed


def kernel(input_ids, word_table, pos_table, scale, shift):
    B, S = input_ids.shape
    assert S == SEQ and word_table.shape[1] == D
    idt = input_ids.T.astype(jnp.int32)
    post = pos_table.T
    out = _make_sc_embed(B)(idt, word_table, post, scale, shift)
    return jnp.transpose(out, (2, 0, 1))


# speedup vs baseline: 1.1474x; 1.1474x over previous
import functools

import jax
import jax.numpy as jnp
from jax import lax
from jax.experimental import pallas as pl
from jax.experimental.pallas import tpu as pltpu
from jax.experimental.pallas import tpu_sc as plsc

D = 64
SEQ = 200
L = 16
NV = D // L
NTW = 2
NBW = 16
TPW = SEQ // NTW
CW = 4096 // NBW


def _hsum16(v):
    lanes = lax.iota(jnp.int32, L)
    dnums = lax.GatherDimensionNumbers(
        offset_dims=(), collapsed_slice_dims=(0,), start_index_map=(0,))
    for sh in (8, 4, 2, 1):
        perm = lax.bitwise_xor(lanes, jnp.int32(sh))
        v = v + lax.gather(v, perm[:, None], dnums, slice_sizes=(1,),
                           mode=lax.GatherScatterMode.PROMISE_IN_BOUNDS)
    return v


def _rsqrt16(v):
    i = lax.bitcast_convert_type(v, jnp.int32)
    i = jnp.int32(0x5F3759DF) - lax.shift_right_logical(i, 1)
    y = lax.bitcast_convert_type(i, jnp.float32)
    for _ in range(2):
        y = y * (1.5 - 0.5 * v * y * y)
    return y


@functools.lru_cache(maxsize=None)
def _make_sc_embed(batch):
    mesh = plsc.VectorSubcoreMesh(core_axis_name="c", subcore_axis_name="s")

    @functools.partial(
        pl.kernel,
        out_type=jax.ShapeDtypeStruct((SEQ, D, batch), jnp.float32),
        mesh=mesh,
        compiler_params=pltpu.CompilerParams(
            use_tc_tiling_on_sc=False, needs_layout_passes=False),
        scratch_types=[
            pltpu.VMEM((CW,), jnp.int32),
            pltpu.VMEM((CW,), jnp.int32),
            pltpu.VMEM((D, 512), jnp.float32),
            pltpu.VMEM((D,), jnp.float32),
            pltpu.VMEM((D,), jnp.float32),
            pltpu.VMEM((CW, D), jnp.float32),
            pltpu.VMEM((CW, D), jnp.float32),
            pltpu.VMEM((D, CW), jnp.float32),
            pltpu.VMEM((D, CW), jnp.float32),
            pltpu.SemaphoreType.DMA,
            pltpu.SemaphoreType.DMA,
            pltpu.SemaphoreType.DMA,
            pltpu.SemaphoreType.DMA,
            pltpu.SemaphoreType.DMA,
            pltpu.SemaphoreType.DMA,
        ],
    )
    def sc_embed(idt_hbm, wt_hbm, post_hbm, sc_hbm, sh_hbm, out_hbm,
                 ids0_v, ids1_v, post_v, scale_v, shift_v, rows0_v, rows1_v,
                 outb0_v, outb1_v, isem0, isem1, gsem0, gsem1, osem0, osem1):
        wid = lax.axis_index("s") * 2 + lax.axis_index("c")
        t0 = (wid // NBW) * TPW
        bb = pl.multiple_of((wid % NBW) * CW, CW)
        pltpu.sync_copy(post_hbm, post_v)
        pltpu.sync_copy(sc_hbm, scale_v)
        pltpu.sync_copy(sh_hbm, shift_v)

        scs = [scale_v[pl.ds(q * L, L)] for q in range(NV)]
        shs = [shift_v[pl.ds(q * L, L)] for q in range(NV)]
        ids = (ids0_v, ids1_v)
        isems = (isem0, isem1)
        gsems = (gsem0, gsem1)
        osems = (osem0, osem1)
        rows = (rows0_v, rows1_v)
        outbs = (outb0_v, outb1_v)

        def start_ids(tt, b):
            pltpu.async_copy(
                idt_hbm.at[t0 + tt, pl.ds(bb, CW)], ids[b], isems[b])

        def drain_ids(b):
            pltpu.make_async_copy(
                idt_hbm.at[0, pl.ds(bb, CW)], ids[b], isems[b]).wait()

        def start_gather(tt, b):
            pltpu.async_copy(wt_hbm.at[ids[b]], rows[b], gsems[b])

        def drain_gather(b):
            pltpu.make_async_copy(
                wt_hbm.at[pl.ds(0, CW)], rows[b], gsems[b]).wait()

        def start_out(tt, b):
            pltpu.async_copy(
                outbs[b], out_hbm.at[t0 + tt, :, pl.ds(bb, CW)], osems[b])

        def drain_out(b):
            pltpu.make_async_copy(
                out_hbm.at[0, :, pl.ds(bb, CW)], outbs[b], osems[b]).wait()

        def compute(tt, b):
            tv = lax.broadcast(t0 + tt, (L,))
            lanes = lax.iota(jnp.int32, L)
            pe = [plsc.load_gather(post_v, [q * L + lanes, tv])
                  for q in range(NV)]

            @plsc.parallel_loop(0, CW, step=1, unroll=8)
            def _row(j):
                e = [rows[b][j, pl.ds(q * L, L)] + pe[q]
                     for q in range(NV)]
                s = (e[0] + e[1]) + (e[2] + e[3])
                q2 = ((e[0] * e[0] + e[1] * e[1])
                      + (e[2] * e[2] + e[3] * e[3]))
                mu = _hsum16(s) * (1.0 / D)
                exx = _hsum16(q2) * (1.0 / D)
                y = _rsqrt16(exx - mu * mu + 1e-12)
                jv = lax.broadcast(j, (L,))
                ln = lax.iota(jnp.int32, L)
                for q in range(NV):
                    plsc.store_scatter(
                        outbs[b], [q * L + ln, jv],
                        ((e[q] - mu) * y) * scs[q] + shs[q])

        def body(tt, b, first, last, penult=False):
            drain_gather(b)
            if not last:
                if not penult:
                    start_ids(tt + 2, b)
                drain_ids(1 - b)
                start_gather(tt + 1, 1 - b)
            if not first:
                drain_out(b)
            compute(tt, b)
            start_out(tt, b)

        start_ids(0, 0)
        drain_ids(0)
        start_gather(0, 0)
        start_ids(1, 1)
        body(0, 0, True, False)
        body(1, 1, True, False)

        def main(s, carry):
            tt = 2 * s
            body(tt, 0, False, False)
            body(tt + 1, 1, False, False)
            return carry

        lax.fori_loop(1, TPW // 2 - 1, main, 0)
        body(TPW - 2, 0, False, False, penult=True)
        body(TPW - 1, 1, False, True)
        drain_out(0)
        drain_out(1)

    return sc_embed


def kernel(input_ids, word_table, pos_table, scale, shift):
    B, S = input_ids.shape
    assert S == SEQ and word_table.shape[1] == D
    idt = input_ids.T.astype(jnp.int32)
    post = pos_table.T
    out = _make_sc_embed(B)(idt, word_table, post, scale, shift)
    return jnp.transpose(out, (2, 0, 1))
